# baseline (speedup 1.0000x reference)
import functools

import jax
import jax.numpy as jnp
from jax import lax
from jax.experimental import pallas as pl
from jax.experimental.pallas import tpu as pltpu
from jax.experimental.pallas import tpu_sc as plsc

L = 16
NC = 2
NS = 16
NW = NC * NS
BATCH_SIZE = 16384
N_FACT = 64
CHUNK = 128
N_CHUNKS = BATCH_SIZE // NW // CHUNK
ROWS_PER_W = CHUNK * N_CHUNKS


def _hsum16(vecs):
    iota = lax.iota(jnp.int32, L)
    cur = list(vecs)
    h = L // 2
    while h >= 1:
        perm = iota ^ h
        mask = (iota & h) == 0
        half = len(cur) // 2
        nxt = []
        for k in range(half):
            x, y = cur[k], cur[k + half]
            x2 = x + x.at[perm].get(mode="promise_in_bounds", unique_indices=True)
            y2 = y + y.at[perm].get(mode="promise_in_bounds", unique_indices=True)
            nxt.append(jnp.where(mask, x2, y2))
        cur = nxt
        h //= 2
    return cur[0]


def _sc_kernel(uf_hbm, if_hbm, uidx_hbm, iidx_hbm, out_hbm,
               uidx_v, iidx_v,
               ub0, ub1, ub2, ub3,
               vb0, vb1, vb2, vb3,
               out_v,
               su0, su1, su2, su3,
               sv0, sv1, sv2, sv3):
    wid = lax.axis_index("s") * NC + lax.axis_index("c")

    pltpu.sync_copy(uidx_hbm.at[wid], uidx_v)
    pltpu.sync_copy(iidx_hbm.at[wid], iidx_v)

    ubufs = [ub0, ub1, ub2, ub3]
    vbufs = [vb0, vb1, vb2, vb3]
    usems = [su0, su1, su2, su3]
    vsems = [sv0, sv1, sv2, sv3]

    ucps = []
    vcps = []
    for c in range(N_CHUNKS):
        ucps.append(pltpu.async_copy(uf_hbm.at[uidx_v.at[c]], ubufs[c], usems[c]))
        vcps.append(pltpu.async_copy(if_hbm.at[iidx_v.at[c]], vbufs[c], vsems[c]))

    for c in range(N_CHUNKS):
        ucps[c].wait()
        vcps[c].wait()
        ub = ubufs[c]
        vb = vbufs[c]

        def body(g, _, ub=ub, vb=vb, c=c):
            partials = []
            for k in range(L):
                r = g * L + k
                acc = ub[r, pl.ds(0, L)] * vb[r, pl.ds(0, L)]
                for j in range(1, N_FACT // L):
                    acc = acc + ub[r, pl.ds(j * L, L)] * vb[r, pl.ds(j * L, L)]
                partials.append(acc)
            out_v[pl.ds(c * CHUNK + g * L, L)] = _hsum16(partials)
            return 0

        lax.fori_loop(0, CHUNK // L, body, 0)

    pltpu.sync_copy(out_v, out_hbm.at[pl.ds(wid * ROWS_PER_W, ROWS_PER_W)])


@functools.partial(
    pl.kernel,
    out_type=jax.ShapeDtypeStruct((BATCH_SIZE,), jnp.float32),
    mesh=plsc.VectorSubcoreMesh(core_axis_name="c", subcore_axis_name="s"),
    compiler_params=pltpu.CompilerParams(use_tc_tiling_on_sc=False),
    scratch_types=(
        [pltpu.VMEM((N_CHUNKS, CHUNK), jnp.int32)] * 2
        + [pltpu.VMEM((CHUNK, N_FACT), jnp.float32)] * (2 * N_CHUNKS)
        + [pltpu.VMEM((ROWS_PER_W,), jnp.float32)]
        + [pltpu.SemaphoreType.DMA] * (2 * N_CHUNKS)
    ),
)
def _mf_dot(uf_hbm, if_hbm, uidx_hbm, iidx_hbm, out_hbm, *scratch):
    _sc_kernel(uf_hbm, if_hbm, uidx_hbm, iidx_hbm, out_hbm, *scratch)


def kernel(user, item, user_factors, item_factors):
    uidx = user.astype(jnp.int32).reshape(NW, N_CHUNKS, CHUNK)
    iidx = item.astype(jnp.int32).reshape(NW, N_CHUNKS, CHUNK)
    return _mf_dot(user_factors, item_factors, uidx, iidx)


# speedup vs baseline: 1.5695x; 1.5695x over previous
import functools

import jax
import jax.numpy as jnp
from jax import lax
from jax.experimental import pallas as pl
from jax.experimental.pallas import tpu as pltpu
from jax.experimental.pallas import tpu_sc as plsc

L = 16
NC = 2
NS = 16
NW = NC * NS
BATCH_SIZE = 16384
N_FACT = 64
ROWS_PER_W = BATCH_SIZE // NW
N_GROUPS = ROWS_PER_W // L


def _hsum16(vecs):
    iota = lax.iota(jnp.int32, L)
    cur = list(vecs)
    h = L // 2
    while h >= 1:
        perm = iota ^ h
        mask = (iota & h) == 0
        half = len(cur) // 2
        nxt = []
        for k in range(half):
            x, y = cur[k], cur[k + half]
            x2 = x + x.at[perm].get(mode="promise_in_bounds", unique_indices=True)
            y2 = y + y.at[perm].get(mode="promise_in_bounds", unique_indices=True)
            nxt.append(jnp.where(mask, x2, y2))
        cur = nxt
        h //= 2
    return cur[0]


def _sc_kernel(uf_hbm, if_hbm, uidx_hbm, iidx_hbm, out_hbm,
               uidx_v, iidx_v, ub, vb, out_v, sem0, sem1):
    wid = lax.axis_index("s") * NC + lax.axis_index("c")
    base = wid * ROWS_PER_W

    pltpu.sync_copy(uidx_hbm.at[pl.ds(base, ROWS_PER_W)], uidx_v)
    pltpu.sync_copy(iidx_hbm.at[pl.ds(base, ROWS_PER_W)], iidx_v)

    sems = [sem0, sem1]

    def fetch(g, buf):
        uvec = uidx_v[pl.ds(g * L, L)]
        ivec = iidx_v[pl.ds(g * L, L)]
        for k in range(L):
            pltpu.async_copy(uf_hbm.at[uvec[k]], ub.at[buf * L + k], sems[buf])
        for k in range(L):
            pltpu.async_copy(if_hbm.at[ivec[k]], vb.at[buf * L + k], sems[buf])

    def drain(buf):
        for k in range(L):
            pltpu.make_async_copy(uf_hbm.at[0], ub.at[buf * L + k],
                                  sems[buf]).wait()
        for k in range(L):
            pltpu.make_async_copy(if_hbm.at[0], vb.at[buf * L + k],
                                  sems[buf]).wait()

    def compute(g, buf):
        partials = []
        for k in range(L):
            o = buf * L + k
            acc = ub[o, pl.ds(0, L)] * vb[o, pl.ds(0, L)]
            for j in range(1, N_FACT // L):
                acc = acc + ub[o, pl.ds(j * L, L)] * vb[o, pl.ds(j * L, L)]
            partials.append(acc)
        out_v[pl.ds(g * L, L)] = _hsum16(partials)

    fetch(0, 0)

    def body(i, _):
        g0 = 2 * i
        fetch(g0 + 1, 1)
        drain(0)
        compute(g0, 0)

        @pl.when(g0 + 2 < N_GROUPS)
        def _():
            fetch(g0 + 2, 0)

        drain(1)
        compute(g0 + 1, 1)
        return 0

    lax.fori_loop(0, N_GROUPS // 2, body, 0)

    pltpu.sync_copy(out_v, out_hbm.at[pl.ds(base, ROWS_PER_W)])


@functools.partial(
    pl.kernel,
    out_type=jax.ShapeDtypeStruct((BATCH_SIZE,), jnp.float32),
    mesh=plsc.VectorSubcoreMesh(core_axis_name="c", subcore_axis_name="s"),
    compiler_params=pltpu.CompilerParams(use_tc_tiling_on_sc=True),
    scratch_types=(
        [pltpu.VMEM((ROWS_PER_W,), jnp.int32)] * 2
        + [pltpu.VMEM((2 * L, N_FACT), jnp.float32)] * 2
        + [pltpu.VMEM((ROWS_PER_W,), jnp.float32)]
        + [pltpu.SemaphoreType.DMA] * 2
    ),
)
def _mf_dot(uf_hbm, if_hbm, uidx_hbm, iidx_hbm, out_hbm, *scratch):
    _sc_kernel(uf_hbm, if_hbm, uidx_hbm, iidx_hbm, out_hbm, *scratch)


def kernel(user, item, user_factors, item_factors):
    return _mf_dot(user_factors, item_factors,
                   user.astype(jnp.int32), item.astype(jnp.int32))
